# baseline (speedup 1.0000x reference)
import jax
import jax.numpy as jnp
from jax import lax
from jax.experimental import pallas as pl
from jax.experimental.pallas import tpu as pltpu, tpu_sc as plsc

MAX_D = 10
D = 128
DP = D + 1
NC, NS, L = 2, 16, 16
NW = NC * NS
R = 32


def _make_kernel(n_pad):
    rows_per_w = n_pad // NW
    nchunks = rows_per_w // R
    npairs = nchunks // 2
    mesh = plsc.VectorSubcoreMesh(core_axis_name="c", subcore_axis_name="s")

    def body(idx_hbm, t16_hbm, out_hbm, idx_v, tab_v, outb0, outb1, semw0, semw1):
        iota = lax.iota(jnp.int32, L)
        wid = lax.axis_index("s") * NC + lax.axis_index("c")
        pltpu.sync_copy(t16_hbm, tab_v)
        pltpu.sync_copy(idx_hbm.at[pl.ds(wid * rows_per_w, rows_per_w)], idx_v)

        outb = (outb0, outb1)
        semw = (semw0, semw1)
        wbase = wid * rows_per_w

        def compute_chunk(k, buf):
            for jg in range(R // L):
                x = plsc.load_gather(idx_v, [iota + (k * R + jg * L)])
                row = jnp.where(x > 1000, MAX_D + 1, jnp.minimum(x, MAX_D))
                g0 = row * (D * L) + iota
                zero = lax.shift_right_logical(x, 31)
                rows = iota + jg * L + zero
                for cc in range(D // 2):
                    va = plsc.load_gather(tab_v, [g0 + cc * L])
                    vb = plsc.load_gather(tab_v, [g0 + (cc + D // 2) * L])
                    plsc.store_scatter(outb[buf], [rows, zero + cc], va)
                    plsc.store_scatter(outb[buf], [rows, zero + (cc + D // 2)], vb)

        def write_chunk(k, buf):
            off = pl.multiple_of((wbase + k * R), R)
            pltpu.async_copy(
                outb[buf].at[:, pl.ds(0, D)],
                out_hbm.at[pl.ds(off, R)],
                semw[buf],
            )

        def wait_chunk(buf):
            pltpu.make_async_copy(
                outb[buf].at[:, pl.ds(0, D)], out_hbm.at[pl.ds(0, R)], semw[buf]
            ).wait()

        def pair(p, carry):
            for b in range(2):
                k = p * 2 + b

                @pl.when(p > 0)
                def _():
                    wait_chunk(b)

                compute_chunk(k, b)
                write_chunk(k, b)
            return carry

        lax.fori_loop(0, npairs, pair, 0)
        wait_chunk(0)
        wait_chunk(1)

    return pl.kernel(
        body,
        out_type=jax.ShapeDtypeStruct((n_pad, D), jnp.float32),
        mesh=mesh,
        compiler_params=pltpu.CompilerParams(needs_layout_passes=False),
        scratch_types=[
            pltpu.VMEM((rows_per_w,), jnp.int32),
            pltpu.VMEM(((MAX_D + 2) * D * L,), jnp.float32),
            pltpu.VMEM((R, DP), jnp.float32),
            pltpu.VMEM((R, DP), jnp.float32),
            pltpu.SemaphoreType.DMA,
            pltpu.SemaphoreType.DMA,
        ],
    )


@jax.jit
def kernel(data, embed_weight):
    n = data.shape[0]
    grain = NW * R * 2
    n_pad = -(-n // grain) * grain
    idx = jnp.reshape(data, (-1,)).astype(jnp.int32)
    idx = jnp.pad(idx, (0, n_pad - n))
    t16 = jnp.broadcast_to(
        jnp.reshape(embed_weight, (-1, 1)), (embed_weight.size, L)
    ).reshape(-1)
    out = _make_kernel(n_pad)(idx, t16)
    return out[:n]


# speedup vs baseline: 1.4055x; 1.4055x over previous
import jax
import jax.numpy as jnp
from jax import lax
from jax.experimental import pallas as pl
from jax.experimental.pallas import tpu as pltpu, tpu_sc as plsc

MAX_D = 10
D = 128
NC, NS, L = 2, 16, 16
NW = NC * NS
R = 32


def _make_kernel(n_pad):
    rows_per_w = n_pad // NW
    nchunks = rows_per_w // R
    npairs = nchunks // 2
    mesh = plsc.VectorSubcoreMesh(core_axis_name="c", subcore_axis_name="s")

    def body(idx_hbm, t16_hbm, out_hbm, idx_v, tab_v, outb0, outb1, semw0, semw1):
        iota = lax.iota(jnp.int32, L)
        wid = lax.axis_index("s") * NC + lax.axis_index("c")
        pltpu.sync_copy(t16_hbm, tab_v)
        pltpu.sync_copy(idx_hbm.at[pl.ds(wid * rows_per_w, rows_per_w)], idx_v)

        outb = (outb0, outb1)
        semw = (semw0, semw1)
        wbase = wid * rows_per_w

        def compute_chunk(k, buf):
            for jg in range(R // L):
                x = plsc.load_gather(idx_v, [iota + (k * R + jg * L)])
                row = jnp.where(x > 1000, MAX_D + 1, jnp.minimum(x, MAX_D))
                zero = lax.shift_right_logical(x, 31)
                g0 = row * (D * L) + iota
                s0 = (iota + jg * L) * D + zero
                ci = iota + zero
                for c in range(D):
                    colv = (ci + c) & (D - 1)
                    cs = colv * L
                    vals = plsc.load_gather(tab_v, [g0 + cs])
                    plsc.store_scatter(outb[buf], [s0 + colv], vals)

        def write_chunk(k, buf):
            off = pl.multiple_of((wbase + k * R) * D, R * D)
            pltpu.async_copy(outb[buf], out_hbm.at[pl.ds(off, R * D)], semw[buf])

        def wait_chunk(buf):
            pltpu.make_async_copy(
                outb[buf], out_hbm.at[pl.ds(0, R * D)], semw[buf]
            ).wait()

        def pair(p, carry):
            for b in range(2):
                k = p * 2 + b

                @pl.when(p > 0)
                def _():
                    wait_chunk(b)

                compute_chunk(k, b)
                write_chunk(k, b)
            return carry

        lax.fori_loop(0, npairs, pair, 0)
        wait_chunk(0)
        wait_chunk(1)

    return pl.kernel(
        body,
        out_type=jax.ShapeDtypeStruct((n_pad * D,), jnp.float32),
        mesh=mesh,
        compiler_params=pltpu.CompilerParams(needs_layout_passes=False),
        scratch_types=[
            pltpu.VMEM((rows_per_w,), jnp.int32),
            pltpu.VMEM(((MAX_D + 2) * D * L,), jnp.float32),
            pltpu.VMEM((R * D,), jnp.float32),
            pltpu.VMEM((R * D,), jnp.float32),
            pltpu.SemaphoreType.DMA,
            pltpu.SemaphoreType.DMA,
        ],
    )


@jax.jit
def kernel(data, embed_weight):
    n = data.shape[0]
    grain = NW * R * 2
    n_pad = -(-n // grain) * grain
    idx = jnp.reshape(data, (-1,)).astype(jnp.int32)
    idx = jnp.pad(idx, (0, n_pad - n))
    t16 = jnp.broadcast_to(
        jnp.reshape(embed_weight, (-1, 1)), (embed_weight.size, L)
    ).reshape(-1)
    out = _make_kernel(n_pad)(idx, t16)
    return jnp.reshape(out, (n_pad, D))[:n]
